# baseline (speedup 1.0000x reference)
import jax
import jax.numpy as jnp
from jax.experimental import pallas as pl

SEQ_LEN = 8192
OUT_DIM = 1024
BATCH = 4
BLOCK_SEQ = 512


def _add_kernel(in_ref, pos_ref, out_ref):
    out_ref[...] = in_ref[...] + pos_ref[...][None, :, :]


def kernel(inputs, pos_table):
    n_seq = SEQ_LEN // BLOCK_SEQ
    return pl.pallas_call(
        _add_kernel,
        grid=(n_seq, BATCH // 2),
        in_specs=[
            pl.BlockSpec((2, BLOCK_SEQ, OUT_DIM), lambda i, b: (b, i, 0)),
            pl.BlockSpec((BLOCK_SEQ, OUT_DIM), lambda i, b: (i, 0)),
        ],
        out_specs=pl.BlockSpec((2, BLOCK_SEQ, OUT_DIM), lambda i, b: (b, i, 0)),
        out_shape=jax.ShapeDtypeStruct((BATCH, SEQ_LEN, OUT_DIM), inputs.dtype),
    )(inputs, pos_table)


# speedup vs baseline: 1.0267x; 1.0267x over previous
import jax
import jax.numpy as jnp
from jax.experimental import pallas as pl

SEQ_LEN = 8192
OUT_DIM = 1024
BATCH = 4
BLOCK_SEQ = 2048


def _add_kernel(in_ref, pos_ref, out_ref):
    out_ref[...] = in_ref[...] + pos_ref[...][None, :, :]


def kernel(inputs, pos_table):
    n_seq = SEQ_LEN // BLOCK_SEQ
    return pl.pallas_call(
        _add_kernel,
        grid=(n_seq, BATCH),
        in_specs=[
            pl.BlockSpec((1, BLOCK_SEQ, OUT_DIM), lambda i, b: (b, i, 0)),
            pl.BlockSpec((BLOCK_SEQ, OUT_DIM), lambda i, b: (i, 0)),
        ],
        out_specs=pl.BlockSpec((1, BLOCK_SEQ, OUT_DIM), lambda i, b: (b, i, 0)),
        out_shape=jax.ShapeDtypeStruct((BATCH, SEQ_LEN, OUT_DIM), inputs.dtype),
    )(inputs, pos_table)
